# baseline (speedup 1.0000x reference)
import jax
import jax.numpy as jnp
from jax import lax
from jax.experimental import pallas as pl
from jax.experimental.pallas import tpu as pltpu
from jax.experimental.pallas import tpu_sc as plsc

_USER_NUM = 1000000
_ITEM_NUM = 1000000
_C0 = 10000
_C1 = 100
_D = 32
_BATCH = 16384

_NC = 2
_NS = 16
_NW = _NC * _NS
_BPW = _BATCH // _NW
_CHUNK = 128
_NCHUNK = _BPW // _CHUNK
_L = 16
_NGROUP = _BPW // _L


def _sc_body(uids_hbm, iids_hbm,
             user_obj, user_c0, user_c1,
             item_obj, item_c0, item_c1,
             out_hbm,
             uid_v, iid_v, uc0_v, uc1_v, ic0_v, ic1_v,
             uo_r, uc0_r, uc1_r, io_r, ic0_r, ic1_r,
             out_v, sem):
    wid = lax.axis_index("s") * _NC + lax.axis_index("c")
    base = wid * _BPW

    pltpu.sync_copy(uids_hbm.at[pl.ds(base, _BPW)], uid_v)
    pltpu.sync_copy(iids_hbm.at[pl.ds(base, _BPW)], iid_v)

    def _mod_body(g, _):
        sl = pl.ds(g * _L, _L)
        u = uid_v[sl]
        i = iid_v[sl]
        uc0_v[sl] = lax.rem(u, _C0)
        uc1_v[sl] = lax.rem(u, _C1)
        ic0_v[sl] = lax.rem(i, _C0)
        ic1_v[sl] = lax.rem(i, _C1)
        return 0

    lax.fori_loop(0, _NGROUP, _mod_body, 0)

    copies = []
    for tab, idx, dst in (
        (user_obj, uid_v, uo_r),
        (user_c0, uc0_v, uc0_r),
        (user_c1, uc1_v, uc1_r),
        (item_obj, iid_v, io_r),
        (item_c0, ic0_v, ic0_r),
        (item_c1, ic1_v, ic1_r),
    ):
        for c in range(_NCHUNK):
            sl = pl.ds(c * _CHUNK, _CHUNK)
            copies.append(
                pltpu.make_async_copy(tab.at[idx.at[sl]], dst.at[sl], sem))
    for cp in copies:
        cp.start()
    for cp in copies:
        cp.wait()

    lanes = lax.iota(jnp.int32, _L)

    def _dot_body(g, _):
        acc = jnp.zeros((_L,), jnp.float32)
        for r16 in range(_L):
            r = g * _L + r16
            p = jnp.zeros((_L,), jnp.float32)
            for h in range(_D // _L):
                sl = pl.ds(h * _L, _L)
                u = uo_r[r, sl] + uc0_r[r, sl] + uc1_r[r, sl]
                v = io_r[r, sl] + ic0_r[r, sl] + ic1_r[r, sl]
                p = p + u * v
            acc = jnp.where(lanes == r16, jnp.sum(p), acc)
        out_v[pl.ds(g * _L, _L)] = acc
        return 0

    lax.fori_loop(0, _NGROUP, _dot_body, 0)

    pltpu.sync_copy(out_v, out_hbm.at[pl.ds(base, _BPW)])


def kernel(X, user_obj, user_c0, user_c1, item_obj, item_c0, item_c1):
    uids = X[:, 0]
    iids = X[:, 1]

    mesh = plsc.VectorSubcoreMesh(core_axis_name="c", subcore_axis_name="s")
    k = pl.kernel(
        _sc_body,
        out_type=jax.ShapeDtypeStruct((_BATCH,), jnp.float32),
        mesh=mesh,
        compiler_params=pltpu.CompilerParams(
            needs_layout_passes=False, use_tc_tiling_on_sc=False),
        scratch_types=[
            pltpu.VMEM((_BPW,), jnp.int32),
            pltpu.VMEM((_BPW,), jnp.int32),
            pltpu.VMEM((_BPW,), jnp.int32),
            pltpu.VMEM((_BPW,), jnp.int32),
            pltpu.VMEM((_BPW,), jnp.int32),
            pltpu.VMEM((_BPW,), jnp.int32),
            pltpu.VMEM((_BPW, _D), jnp.float32),
            pltpu.VMEM((_BPW, _D), jnp.float32),
            pltpu.VMEM((_BPW, _D), jnp.float32),
            pltpu.VMEM((_BPW, _D), jnp.float32),
            pltpu.VMEM((_BPW, _D), jnp.float32),
            pltpu.VMEM((_BPW, _D), jnp.float32),
            pltpu.VMEM((_BPW,), jnp.float32),
            pltpu.SemaphoreType.DMA,
        ],
    )
    out = k(uids, iids, user_obj, user_c0, user_c1,
            item_obj, item_c0, item_c1)
    return out.reshape(_BATCH, 1)
